# Initial kernel scaffold; baseline (speedup 1.0000x reference)
import jax
import jax.numpy as jnp
from jax.experimental import pallas as pl


def kernel(inputs, graph_indices, W_ih, W_hh, b):
    raise NotImplementedError("write your pallas kernel here")


# speedup vs baseline: 7.6980x; 7.6980x over previous
import functools

import jax
import jax.numpy as jnp
from jax.experimental import pallas as pl
from jax.experimental.pallas import tpu as pltpu

_UNITS = 128
_STEPS = 3
_NUM_GRAPHS = 64
_BLOCK = 2000


def _dot(a, b, dims):
    return jax.lax.dot_general(
        a, b, (dims, ((), ())),
        precision=jax.lax.Precision.HIGHEST,
        preferred_element_type=jnp.float32)


def _body(x_ref, g_ref, wih_ref, whh_ref, b_ref, out_ref,
          h_s, c_s, q_s, m_s, l_s, r_s, *, nblocks):
    t = pl.program_id(0)
    i = pl.program_id(1)
    G = _NUM_GRAPHS
    U = _UNITS

    @pl.when(i == 0)
    def _start_step():
        @pl.when(t == 0)
        def _init():
            h_s[...] = jnp.zeros_like(h_s)
            c_s[...] = jnp.zeros_like(c_s)
            q_s[...] = jnp.zeros_like(q_s)

        gates = (_dot(q_s[...], wih_ref[...], (((1,), (0,))))
                 + _dot(h_s[...], whh_ref[...], (((1,), (0,))))
                 + b_ref[...])
        i_g = jax.nn.sigmoid(gates[:, :U])
        f_g = jax.nn.sigmoid(gates[:, U:2 * U])
        g_g = jnp.tanh(gates[:, 2 * U:3 * U])
        o_g = jax.nn.sigmoid(gates[:, 3 * U:])
        c_new = f_g * c_s[...] + i_g * g_g
        h_s[...] = o_g * jnp.tanh(c_new)
        c_s[...] = c_new
        m_s[...] = jnp.full_like(m_s, -jnp.inf)
        l_s[...] = jnp.zeros_like(l_s)
        r_s[...] = jnp.zeros_like(r_s)

    x = x_ref[...]
    g = g_ref[0]
    h = h_s[...]

    e = _dot(h, x, (((1,), (1,))))
    mask = jax.lax.broadcasted_iota(jnp.int32, e.shape, 0) == g

    m_old = m_s[...]
    m_blk = jnp.max(jnp.where(mask, e, -jnp.inf), axis=1, keepdims=True)
    m_new = jnp.maximum(m_old, m_blk)
    alpha = jnp.where(m_new == -jnp.inf, 0.0, jnp.exp(m_old - m_new))

    e_node = jnp.sum(jnp.where(mask, e, 0.0), axis=0, keepdims=True)
    m_node = jnp.sum(jnp.where(mask, m_new, 0.0), axis=0, keepdims=True)
    p = jnp.where(mask, jnp.exp(e_node - m_node), 0.0)

    l_s[...] = alpha * l_s[...] + jnp.sum(p, axis=1, keepdims=True)
    r_s[...] = alpha * r_s[...] + _dot(p, x, (((1,), (0,))))
    m_s[...] = m_new

    @pl.when(i == nblocks - 1)
    def _end_step():
        l = l_s[...]
        r = jnp.where(l > 0.0, r_s[...] / l, 0.0)
        q_star = jnp.concatenate([h_s[...], r], axis=1)
        q_s[...] = q_star
        out_ref[...] = q_star


def kernel(inputs, graph_indices, W_ih, W_hh, b):
    n, u = inputs.shape
    assert u == _UNITS
    B = _BLOCK
    if n % B:
        pad = B - n % B
        inputs = jnp.pad(inputs, ((0, pad), (0, 0)))
        graph_indices = jnp.pad(graph_indices, (0, pad),
                                constant_values=_NUM_GRAPHS)
        n += pad
    nblocks = n // B
    g3 = jnp.asarray(graph_indices, jnp.int32).reshape(nblocks, 1, B)
    b2 = jnp.asarray(b, jnp.float32).reshape(1, 4 * u)

    G = _NUM_GRAPHS
    grid = (_STEPS, nblocks)
    return pl.pallas_call(
        functools.partial(_body, nblocks=nblocks),
        grid=grid,
        in_specs=[
            pl.BlockSpec((B, u), lambda t, i: (i, 0)),
            pl.BlockSpec((1, 1, B), lambda t, i: (i, 0, 0)),
            pl.BlockSpec((2 * u, 4 * u), lambda t, i: (0, 0)),
            pl.BlockSpec((u, 4 * u), lambda t, i: (0, 0)),
            pl.BlockSpec((1, 4 * u), lambda t, i: (0, 0)),
        ],
        out_specs=pl.BlockSpec((G, 2 * u), lambda t, i: (0, 0)),
        out_shape=jax.ShapeDtypeStruct((G, 2 * u), jnp.float32),
        scratch_shapes=[
            pltpu.VMEM((G, u), jnp.float32),
            pltpu.VMEM((G, u), jnp.float32),
            pltpu.VMEM((G, 2 * u), jnp.float32),
            pltpu.VMEM((G, 1), jnp.float32),
            pltpu.VMEM((G, 1), jnp.float32),
            pltpu.VMEM((G, u), jnp.float32),
        ],
    )(inputs, g3, W_ih, W_hh, b2)


# speedup vs baseline: 16.4705x; 2.1396x over previous
import functools

import jax
import jax.numpy as jnp
from jax.experimental import pallas as pl
from jax.experimental.pallas import tpu as pltpu

_UNITS = 128
_STEPS = 3
_NUM_GRAPHS = 64
_BLOCK = 2000


def _dot(a, b, dims, precision=jax.lax.Precision.DEFAULT):
    return jax.lax.dot_general(
        a, b, (dims, ((), ())),
        precision=precision,
        preferred_element_type=jnp.float32)


def _body(x_ref, g_ref, wih_ref, whh_ref, b_ref, out_ref,
          h_s, c_s, q_s, m_s, l_s, r_s, *, nblocks):
    t = pl.program_id(0)
    i = pl.program_id(1)
    G = _NUM_GRAPHS
    U = _UNITS

    @pl.when(i == 0)
    def _start_step():
        @pl.when(t == 0)
        def _init():
            h_s[...] = jnp.zeros_like(h_s)
            c_s[...] = jnp.zeros_like(c_s)
            q_s[...] = jnp.zeros_like(q_s)

        hi = jax.lax.Precision.HIGHEST
        gates = (_dot(q_s[...], wih_ref[...], (((1,), (0,))), hi)
                 + _dot(h_s[...], whh_ref[...], (((1,), (0,))), hi)
                 + b_ref[...])
        i_g = jax.nn.sigmoid(gates[:, :U])
        f_g = jax.nn.sigmoid(gates[:, U:2 * U])
        g_g = jnp.tanh(gates[:, 2 * U:3 * U])
        o_g = jax.nn.sigmoid(gates[:, 3 * U:])
        c_new = f_g * c_s[...] + i_g * g_g
        h_s[...] = o_g * jnp.tanh(c_new)
        c_s[...] = c_new
        m_s[...] = jnp.full_like(m_s, -jnp.inf)
        l_s[...] = jnp.zeros_like(l_s)
        r_s[...] = jnp.zeros_like(r_s)

    x = x_ref[...]
    g = g_ref[0]
    h = h_s[...]

    e = _dot(h, x, (((1,), (1,))))
    mask = jax.lax.broadcasted_iota(jnp.int32, e.shape, 0) == g

    m_old = m_s[...]
    em = jnp.where(mask, e, -jnp.inf)
    m_blk = jnp.max(em, axis=1, keepdims=True)
    m_new = jnp.maximum(m_old, m_blk)
    m_safe = jnp.where(m_new == -jnp.inf, 0.0, m_new)
    alpha = jnp.exp(m_old - m_safe)
    p = jnp.exp(em - m_safe)

    l_s[...] = alpha * l_s[...] + jnp.sum(p, axis=1, keepdims=True)
    r_s[...] = alpha * r_s[...] + _dot(p, x, (((1,), (0,))))
    m_s[...] = m_new

    @pl.when(i == nblocks - 1)
    def _end_step():
        l = l_s[...]
        r = jnp.where(l > 0.0, r_s[...] / l, 0.0)
        q_star = jnp.concatenate([h_s[...], r], axis=1)
        q_s[...] = q_star
        out_ref[...] = q_star


def kernel(inputs, graph_indices, W_ih, W_hh, b):
    n, u = inputs.shape
    assert u == _UNITS
    B = _BLOCK
    if n % B:
        pad = B - n % B
        inputs = jnp.pad(inputs, ((0, pad), (0, 0)))
        graph_indices = jnp.pad(graph_indices, (0, pad),
                                constant_values=_NUM_GRAPHS)
        n += pad
    nblocks = n // B
    g3 = jnp.asarray(graph_indices, jnp.int32).reshape(nblocks, 1, B)
    b2 = jnp.asarray(b, jnp.float32).reshape(1, 4 * u)

    G = _NUM_GRAPHS
    grid = (_STEPS, nblocks)
    return pl.pallas_call(
        functools.partial(_body, nblocks=nblocks),
        grid=grid,
        in_specs=[
            pl.BlockSpec((B, u), lambda t, i: (i, 0)),
            pl.BlockSpec((1, 1, B), lambda t, i: (i, 0, 0)),
            pl.BlockSpec((2 * u, 4 * u), lambda t, i: (0, 0)),
            pl.BlockSpec((u, 4 * u), lambda t, i: (0, 0)),
            pl.BlockSpec((1, 4 * u), lambda t, i: (0, 0)),
        ],
        out_specs=pl.BlockSpec((G, 2 * u), lambda t, i: (0, 0)),
        out_shape=jax.ShapeDtypeStruct((G, 2 * u), jnp.float32),
        scratch_shapes=[
            pltpu.VMEM((G, u), jnp.float32),
            pltpu.VMEM((G, u), jnp.float32),
            pltpu.VMEM((G, 2 * u), jnp.float32),
            pltpu.VMEM((G, 1), jnp.float32),
            pltpu.VMEM((G, 1), jnp.float32),
            pltpu.VMEM((G, u), jnp.float32),
        ],
    )(inputs, g3, W_ih, W_hh, b2)
